# baseline (speedup 1.0000x reference)
import functools

import jax
import jax.numpy as jnp
from jax import lax
from jax.experimental import pallas as pl
from jax.experimental.pallas import tpu as pltpu
from jax.experimental.pallas import tpu_sc as plsc

LIMIT = 7.0
ACT_ALPHA = 1.702
TBL = 128
NC = 2
NS = 16
NW = NC * NS


def _group_body(te_ref, x_ref, wg_ref, wu_ref, w2_ref, bg_ref, bu_ref, b2_ref,
                out_ref):
    x = x_ref[...]
    gate = jnp.dot(x, wg_ref[0], preferred_element_type=jnp.float32) + bg_ref[0]
    up = jnp.dot(x, wu_ref[0], preferred_element_type=jnp.float32) + bu_ref[0]
    gate = jnp.minimum(gate, LIMIT)
    up = jnp.clip(up, -LIMIT, LIMIT)
    glu = gate * (1.0 / (1.0 + jnp.exp(-ACT_ALPHA * gate)))
    gated = ((up + 1.0) * glu).astype(jnp.bfloat16)
    out_ref[...] = (jnp.dot(gated, w2_ref[0], preferred_element_type=jnp.float32)
                    + b2_ref[0])


def kernel(hidden_states, routing_weights, gate_up_proj, gate_up_proj_bias,
           down_proj, down_proj_bias, lora_gate_up_A, lora_gate_up_B,
           lora_down_A, lora_down_B, router_indices):
    B_SZ, S, H = hidden_states.shape
    E, _, D = gate_up_proj.shape
    F = D // 2
    R = lora_gate_up_A.shape[1] // H
    scaling = 1.0 / R
    T = B_SZ * S
    TOPK = router_indices.shape[1]
    NA = T * TOPK
    NTILES = NA // TBL + E
    NPAD = NTILES * TBL
    H2 = H // 2

    flat_e = router_indices.reshape(NA)
    onehot = (flat_e[:, None] == jnp.arange(E, dtype=jnp.int32)[None, :]
              ).astype(jnp.float32)
    counts = jnp.sum(onehot, axis=0)
    csum = jnp.cumsum(onehot, axis=0)
    rank = jnp.sum((csum - onehot) * onehot, axis=1)
    padded_counts = jnp.ceil(counts / TBL) * TBL
    ends = jnp.cumsum(padded_counts)
    padded_off = ends - padded_counts
    dest = (onehot @ padded_off + rank).astype(jnp.int32)
    dest2 = dest.reshape(T, TOPK)
    d0 = dest2[:, 0]
    d1 = dest2[:, 1]
    rw0 = routing_weights[:, 0]
    rw1 = routing_weights[:, 1]
    tile_start = jnp.arange(NTILES, dtype=jnp.float32)[:, None] * TBL
    tile_expert = jnp.minimum(
        jnp.sum((tile_start >= ends[None, :]).astype(jnp.int32), axis=1),
        E - 1).astype(jnp.int32)

    A1 = lora_gate_up_A.reshape(E, H, R)
    B1 = lora_gate_up_B.reshape(E, R, D)
    w1_eff = (gate_up_proj + jnp.einsum('ehr,erd->ehd', A1, B1,
                                        preferred_element_type=jnp.float32)
              * scaling)
    wg = w1_eff[:, :, 0::2].astype(jnp.bfloat16)
    wu = w1_eff[:, :, 1::2].astype(jnp.bfloat16)
    A2 = lora_down_A.reshape(E, F, R)
    B2 = lora_down_B.reshape(E, R, H)
    w2_eff = (down_proj + jnp.einsum('efr,erh->efh', A2, B2,
                                     preferred_element_type=jnp.float32)
              * scaling).astype(jnp.bfloat16)
    bg = gate_up_proj_bias[:, 0::2].reshape(E, 1, F)
    bu = gate_up_proj_bias[:, 1::2].reshape(E, 1, F)
    b2 = down_proj_bias.reshape(E, 1, H)
    xb = hidden_states.reshape(T, H).astype(jnp.bfloat16)
    x32 = lax.bitcast_convert_type(xb.reshape(T, H2, 2), jnp.int32)

    mesh = plsc.VectorSubcoreMesh(core_axis_name="c", subcore_axis_name="s",
                                  num_cores=NC, num_subcores=NS)
    rows_w = T // NW

    @functools.partial(
        pl.kernel,
        out_type=jax.ShapeDtypeStruct((NPAD, H2), jnp.int32),
        mesh=mesh,
        scratch_types=[
            pltpu.VMEM((rows_w, H2), jnp.int32),
            pltpu.VMEM((rows_w,), jnp.int32),
            pltpu.VMEM((rows_w,), jnp.int32),
            pltpu.SemaphoreType.DMA,
        ],
    )
    def scatter_rows(x_hbm, d0_hbm, d1_hbm, out_hbm, rows_v, i0_v, i1_v, sem):
        wid = lax.axis_index("s") * NC + lax.axis_index("c")
        base = wid * rows_w
        pltpu.sync_copy(x_hbm.at[pl.ds(base, rows_w)], rows_v)
        pltpu.sync_copy(d0_hbm.at[pl.ds(base, rows_w)], i0_v)
        pltpu.sync_copy(d1_hbm.at[pl.ds(base, rows_w)], i1_v)
        c0 = pltpu.async_copy(rows_v, out_hbm.at[i0_v], sem)
        c1 = pltpu.async_copy(rows_v, out_hbm.at[i1_v], sem)
        c0.wait()
        c1.wait()

    xs32 = scatter_rows(x32, d0, d1)
    xs = lax.bitcast_convert_type(xs32, jnp.bfloat16).reshape(NPAD, H)

    grid_spec = pltpu.PrefetchScalarGridSpec(
        num_scalar_prefetch=1,
        grid=(NTILES,),
        in_specs=[
            pl.BlockSpec((TBL, H), lambda i, te: (i, 0)),
            pl.BlockSpec((1, H, F), lambda i, te: (te[i], 0, 0)),
            pl.BlockSpec((1, H, F), lambda i, te: (te[i], 0, 0)),
            pl.BlockSpec((1, F, H), lambda i, te: (te[i], 0, 0)),
            pl.BlockSpec((1, 1, F), lambda i, te: (te[i], 0, 0)),
            pl.BlockSpec((1, 1, F), lambda i, te: (te[i], 0, 0)),
            pl.BlockSpec((1, 1, H), lambda i, te: (te[i], 0, 0)),
        ],
        out_specs=pl.BlockSpec((TBL, H), lambda i, te: (i, 0)),
    )
    contrib = pl.pallas_call(
        _group_body,
        grid_spec=grid_spec,
        out_shape=jax.ShapeDtypeStruct((NPAD, H), jnp.float32),
    )(tile_expert, xs, wg, wu, w2_eff, bg, bu, b2)

    CH = min(32, rows_w)

    @functools.partial(
        pl.kernel,
        out_type=jax.ShapeDtypeStruct((T, H), jnp.float32),
        mesh=mesh,
        scratch_types=[
            pltpu.VMEM((CH, H), jnp.float32),
            pltpu.VMEM((CH, H), jnp.float32),
            pltpu.VMEM((CH,), jnp.int32),
            pltpu.VMEM((CH,), jnp.int32),
            pltpu.VMEM((CH,), jnp.float32),
            pltpu.VMEM((CH,), jnp.float32),
            pltpu.SemaphoreType.DMA,
        ],
    )
    def combine(contrib_hbm, d0_hbm, d1_hbm, w0_hbm, w1_hbm, out_hbm,
                r0_v, r1_v, i0_v, i1_v, w0_v, w1_v, sem):
        wid = lax.axis_index("s") * NC + lax.axis_index("c")
        for c in range(rows_w // CH):
            tb = wid * rows_w + c * CH
            pltpu.sync_copy(d0_hbm.at[pl.ds(tb, CH)], i0_v)
            pltpu.sync_copy(d1_hbm.at[pl.ds(tb, CH)], i1_v)
            pltpu.sync_copy(w0_hbm.at[pl.ds(tb, CH)], w0_v)
            pltpu.sync_copy(w1_hbm.at[pl.ds(tb, CH)], w1_v)
            g0 = pltpu.async_copy(contrib_hbm.at[i0_v], r0_v, sem)
            g1 = pltpu.async_copy(contrib_hbm.at[i1_v], r1_v, sem)
            g0.wait()
            g1.wait()

            for g in range(CH // 16):
                wv0 = w0_v[pl.ds(g * 16, 16)]
                wv1 = w1_v[pl.ds(g * 16, 16)]
                for tl in range(16):
                    t = g * 16 + tl
                    wt0 = wv0[tl]
                    wt1 = wv1[tl]

                    def vec_body(j, _, t=t, wt0=wt0, wt1=wt1):
                        sl = pl.ds(j * 16, 16)
                        r0_v[t, sl] = wt0 * r0_v[t, sl] + wt1 * r1_v[t, sl]
                        return 0

                    lax.fori_loop(0, H // 16, vec_body, 0)
            pltpu.sync_copy(r0_v, out_hbm.at[pl.ds(tb, CH)])

    out = combine(contrib, d0, d1, rw0, rw1)
    return out.reshape(B_SZ, S, H)


# speedup vs baseline: 3.1370x; 3.1370x over previous
import functools

import jax
import jax.numpy as jnp
from jax import lax
from jax.experimental import pallas as pl
from jax.experimental.pallas import tpu as pltpu
from jax.experimental.pallas import tpu_sc as plsc

LIMIT = 7.0
ACT_ALPHA = 1.702
TBL = 128
NC = 2
NS = 16
NW = NC * NS


def _group_body(te_ref, x_ref, w1_ref, w2_ref, b1_ref, b2_ref, out_ref):
    x = x_ref[...]
    gu = jnp.dot(x, w1_ref[0], preferred_element_type=jnp.float32) + b1_ref[0]
    gur = pltpu.roll(gu, gu.shape[1] - 1, 1)
    gate = jnp.minimum(gu, LIMIT)
    glu = gate * (1.0 / (1.0 + jnp.exp(-ACT_ALPHA * gate)))
    up1 = jnp.clip(gur, -LIMIT, LIMIT) + 1.0
    gated = (glu * up1).astype(jnp.bfloat16)
    out_ref[...] = (jnp.dot(gated, w2_ref[0], preferred_element_type=jnp.float32)
                    + b2_ref[0])


def kernel(hidden_states, routing_weights, gate_up_proj, gate_up_proj_bias,
           down_proj, down_proj_bias, lora_gate_up_A, lora_gate_up_B,
           lora_down_A, lora_down_B, router_indices):
    B_SZ, S, H = hidden_states.shape
    E, _, D = gate_up_proj.shape
    F = D // 2
    R = lora_gate_up_A.shape[1] // H
    scaling = 1.0 / R
    T = B_SZ * S
    TOPK = router_indices.shape[1]
    NA = T * TOPK
    NTILES = NA // TBL + E
    NPAD = NTILES * TBL
    H2 = H // 2

    flat_e = router_indices.reshape(NA)
    onehot = (flat_e[:, None] == jnp.arange(E, dtype=jnp.int32)[None, :]
              ).astype(jnp.float32)
    counts = jnp.sum(onehot, axis=0)
    csum = jnp.cumsum(onehot, axis=0)
    rank = jnp.sum((csum - onehot) * onehot, axis=1)
    padded_counts = jnp.ceil(counts / TBL) * TBL
    ends = jnp.cumsum(padded_counts)
    padded_off = ends - padded_counts
    dest = (onehot @ padded_off + rank).astype(jnp.int32)
    dest2 = dest.reshape(T, TOPK)
    d0 = dest2[:, 0]
    d1 = dest2[:, 1]
    rw0 = routing_weights[:, 0]
    rw1 = routing_weights[:, 1]
    tile_start = jnp.arange(NTILES, dtype=jnp.float32)[:, None] * TBL
    tile_expert = jnp.minimum(
        jnp.sum((tile_start >= ends[None, :]).astype(jnp.int32), axis=1),
        E - 1).astype(jnp.int32)

    A1 = lora_gate_up_A.reshape(E, H, R)
    B1 = lora_gate_up_B.reshape(E, R, D)
    w1_eff = (gate_up_proj + jnp.einsum('ehr,erd->ehd', A1, B1,
                                        preferred_element_type=jnp.float32)
              * scaling).astype(jnp.bfloat16)
    A2 = lora_down_A.reshape(E, F, R)
    B2 = lora_down_B.reshape(E, R, H)
    w2_eff = (down_proj + jnp.einsum('efr,erh->efh', A2, B2,
                                     preferred_element_type=jnp.float32)
              * scaling).astype(jnp.bfloat16)
    w2i = jnp.concatenate(
        [w2_eff[:, :, None, :], jnp.zeros_like(w2_eff)[:, :, None, :]],
        axis=2).reshape(E, D, H)
    b1 = gate_up_proj_bias.reshape(E, 1, D)
    b2 = down_proj_bias.reshape(E, 1, H)
    xb = hidden_states.reshape(T, H).astype(jnp.bfloat16)
    x32 = lax.bitcast_convert_type(xb.reshape(T, H2, 2), jnp.int32)

    mesh = plsc.VectorSubcoreMesh(core_axis_name="c", subcore_axis_name="s",
                                  num_cores=NC, num_subcores=NS)
    rows_w = T // NW

    @functools.partial(
        pl.kernel,
        out_type=jax.ShapeDtypeStruct((NPAD, H2), jnp.int32),
        mesh=mesh,
        scratch_types=[
            pltpu.VMEM((rows_w, H2), jnp.int32),
            pltpu.VMEM((rows_w,), jnp.int32),
            pltpu.VMEM((rows_w,), jnp.int32),
            pltpu.SemaphoreType.DMA,
        ],
    )
    def scatter_rows(x_hbm, d0_hbm, d1_hbm, out_hbm, rows_v, i0_v, i1_v, sem):
        wid = lax.axis_index("s") * NC + lax.axis_index("c")
        base = wid * rows_w
        pltpu.sync_copy(x_hbm.at[pl.ds(base, rows_w)], rows_v)
        pltpu.sync_copy(d0_hbm.at[pl.ds(base, rows_w)], i0_v)
        pltpu.sync_copy(d1_hbm.at[pl.ds(base, rows_w)], i1_v)
        c0 = pltpu.async_copy(rows_v, out_hbm.at[i0_v], sem)
        c1 = pltpu.async_copy(rows_v, out_hbm.at[i1_v], sem)
        c0.wait()
        c1.wait()

    xs32 = scatter_rows(x32, d0, d1)
    xs = lax.bitcast_convert_type(xs32, jnp.bfloat16).reshape(NPAD, H)

    grid_spec = pltpu.PrefetchScalarGridSpec(
        num_scalar_prefetch=1,
        grid=(NTILES,),
        in_specs=[
            pl.BlockSpec((TBL, H), lambda i, te: (i, 0)),
            pl.BlockSpec((1, H, D), lambda i, te: (te[i], 0, 0)),
            pl.BlockSpec((1, D, H), lambda i, te: (te[i], 0, 0)),
            pl.BlockSpec((1, 1, D), lambda i, te: (te[i], 0, 0)),
            pl.BlockSpec((1, 1, H), lambda i, te: (te[i], 0, 0)),
        ],
        out_specs=pl.BlockSpec((TBL, H), lambda i, te: (i, 0)),
    )
    contrib = pl.pallas_call(
        _group_body,
        grid_spec=grid_spec,
        out_shape=jax.ShapeDtypeStruct((NPAD, H), jnp.float32),
    )(tile_expert, xs, w1_eff, w2i, b1, b2)

    CH = min(32, rows_w)

    @functools.partial(
        pl.kernel,
        out_type=jax.ShapeDtypeStruct((T, H), jnp.float32),
        mesh=mesh,
        scratch_types=[
            pltpu.VMEM((CH, H), jnp.float32),
            pltpu.VMEM((CH, H), jnp.float32),
            pltpu.VMEM((CH,), jnp.int32),
            pltpu.VMEM((CH,), jnp.int32),
            pltpu.VMEM((CH,), jnp.float32),
            pltpu.VMEM((CH,), jnp.float32),
            pltpu.SemaphoreType.DMA,
        ],
    )
    def combine(contrib_hbm, d0_hbm, d1_hbm, w0_hbm, w1_hbm, out_hbm,
                r0_v, r1_v, i0_v, i1_v, w0_v, w1_v, sem):
        wid = lax.axis_index("s") * NC + lax.axis_index("c")
        for c in range(rows_w // CH):
            tb = wid * rows_w + c * CH
            pltpu.sync_copy(d0_hbm.at[pl.ds(tb, CH)], i0_v)
            pltpu.sync_copy(d1_hbm.at[pl.ds(tb, CH)], i1_v)
            pltpu.sync_copy(w0_hbm.at[pl.ds(tb, CH)], w0_v)
            pltpu.sync_copy(w1_hbm.at[pl.ds(tb, CH)], w1_v)
            g0 = pltpu.async_copy(contrib_hbm.at[i0_v], r0_v, sem)
            g1 = pltpu.async_copy(contrib_hbm.at[i1_v], r1_v, sem)
            g0.wait()
            g1.wait()

            for g in range(CH // 16):
                wv0 = w0_v[pl.ds(g * 16, 16)]
                wv1 = w1_v[pl.ds(g * 16, 16)]
                for tl in range(16):
                    t = g * 16 + tl
                    wt0 = wv0[tl]
                    wt1 = wv1[tl]

                    def vec_body(j, _, t=t, wt0=wt0, wt1=wt1):
                        sl = pl.ds(j * 16, 16)
                        r0_v[t, sl] = wt0 * r0_v[t, sl] + wt1 * r1_v[t, sl]
                        return 0

                    lax.fori_loop(0, H // 16, vec_body, 0)
            pltpu.sync_copy(r0_v, out_hbm.at[pl.ds(tb, CH)])

    out = combine(contrib, d0, d1, rw0, rw1)
    return out.reshape(B_SZ, S, H)


# speedup vs baseline: 6.7792x; 2.1611x over previous
import functools

import jax
import jax.numpy as jnp
from jax import lax
from jax.experimental import pallas as pl
from jax.experimental.pallas import tpu as pltpu
from jax.experimental.pallas import tpu_sc as plsc

LIMIT = 7.0
ACT_ALPHA = 1.702
TBL = 128
NC = 2
NS = 16
NW = NC * NS


def _group_body(te_ref, x_ref, w1_ref, w2_ref, p_ref, b1_ref, b2_ref, out_ref):
    x = x_ref[...].astype(jnp.bfloat16)
    gu = jnp.dot(x, w1_ref[0], preferred_element_type=jnp.float32) + b1_ref[0]
    gur = pltpu.roll(gu, gu.shape[1] - 1, 1)
    gate = jnp.minimum(gu, LIMIT)
    glu = gate * (1.0 / (1.0 + jnp.exp(-ACT_ALPHA * gate)))
    up1 = jnp.clip(gur, -LIMIT, LIMIT) + 1.0
    gated = (glu * up1).astype(jnp.bfloat16)
    gated_c = jnp.dot(gated, p_ref[...],
                      preferred_element_type=jnp.float32).astype(jnp.bfloat16)
    out_ref[...] = (jnp.dot(gated_c, w2_ref[0],
                            preferred_element_type=jnp.float32) + b2_ref[0])


def kernel(hidden_states, routing_weights, gate_up_proj, gate_up_proj_bias,
           down_proj, down_proj_bias, lora_gate_up_A, lora_gate_up_B,
           lora_down_A, lora_down_B, router_indices):
    B_SZ, S, H = hidden_states.shape
    E, _, D = gate_up_proj.shape
    F = D // 2
    R = lora_gate_up_A.shape[1] // H
    scaling = 1.0 / R
    T = B_SZ * S
    TOPK = router_indices.shape[1]
    NA = T * TOPK
    NTILES = NA // TBL + E
    NPAD = NTILES * TBL
    H2 = H // 2

    flat_e = router_indices.reshape(NA)
    onehot = (flat_e[:, None] == jnp.arange(E, dtype=jnp.int32)[None, :]
              ).astype(jnp.float32)
    counts = jnp.sum(onehot, axis=0)
    csum = jnp.cumsum(onehot, axis=0)
    rank = jnp.sum((csum - onehot) * onehot, axis=1)
    padded_counts = jnp.ceil(counts / TBL) * TBL
    ends = jnp.cumsum(padded_counts)
    padded_off = ends - padded_counts
    dest = (onehot @ padded_off + rank).astype(jnp.int32)
    dest2 = dest.reshape(T, TOPK)
    d0 = dest2[:, 0]
    d1 = dest2[:, 1]
    rw0 = routing_weights[:, 0]
    rw1 = routing_weights[:, 1]
    tile_start = jnp.arange(NTILES, dtype=jnp.float32)[:, None] * TBL
    tile_expert = jnp.minimum(
        jnp.sum((tile_start >= ends[None, :]).astype(jnp.int32), axis=1),
        E - 1).astype(jnp.int32)

    A1 = lora_gate_up_A.reshape(E, H, R)
    B1 = lora_gate_up_B.reshape(E, R, D)
    w1_eff = (gate_up_proj + jnp.einsum('ehr,erd->ehd', A1, B1,
                                        preferred_element_type=jnp.float32)
              * scaling).astype(jnp.bfloat16)
    A2 = lora_down_A.reshape(E, F, R)
    B2 = lora_down_B.reshape(E, R, H)
    w2_eff = (down_proj + jnp.einsum('efr,erh->efh', A2, B2,
                                     preferred_element_type=jnp.float32)
              * scaling).astype(jnp.bfloat16)
    psel = (jnp.arange(D, dtype=jnp.int32)[:, None]
            == 2 * jnp.arange(F, dtype=jnp.int32)[None, :]).astype(jnp.bfloat16)
    b1 = gate_up_proj_bias.reshape(E, 1, D)
    b2 = down_proj_bias.reshape(E, 1, H)
    xf = hidden_states.reshape(T, H)

    mesh = plsc.VectorSubcoreMesh(core_axis_name="c", subcore_axis_name="s",
                                  num_cores=NC, num_subcores=NS)
    rows_w = T // NW

    @functools.partial(
        pl.kernel,
        out_type=jax.ShapeDtypeStruct((NPAD, H), jnp.float32),
        mesh=mesh,
        scratch_types=[
            pltpu.VMEM((rows_w, H), jnp.float32),
            pltpu.VMEM((rows_w,), jnp.int32),
            pltpu.VMEM((rows_w,), jnp.int32),
            pltpu.SemaphoreType.DMA,
        ],
    )
    def scatter_rows(x_hbm, d0_hbm, d1_hbm, out_hbm, rows_v, i0_v, i1_v, sem):
        wid = lax.axis_index("s") * NC + lax.axis_index("c")
        base = wid * rows_w
        pltpu.sync_copy(x_hbm.at[pl.ds(base, rows_w)], rows_v)
        pltpu.sync_copy(d0_hbm.at[pl.ds(base, rows_w)], i0_v)
        pltpu.sync_copy(d1_hbm.at[pl.ds(base, rows_w)], i1_v)
        c0 = pltpu.async_copy(rows_v, out_hbm.at[i0_v], sem)
        c1 = pltpu.async_copy(rows_v, out_hbm.at[i1_v], sem)
        c0.wait()
        c1.wait()

    xs = scatter_rows(xf, d0, d1)

    grid_spec = pltpu.PrefetchScalarGridSpec(
        num_scalar_prefetch=1,
        grid=(NTILES,),
        in_specs=[
            pl.BlockSpec((TBL, H), lambda i, te: (i, 0)),
            pl.BlockSpec((1, H, D), lambda i, te: (te[i], 0, 0)),
            pl.BlockSpec((1, F, H), lambda i, te: (te[i], 0, 0)),
            pl.BlockSpec((D, F), lambda i, te: (0, 0)),
            pl.BlockSpec((1, 1, D), lambda i, te: (te[i], 0, 0)),
            pl.BlockSpec((1, 1, H), lambda i, te: (te[i], 0, 0)),
        ],
        out_specs=pl.BlockSpec((TBL, H), lambda i, te: (i, 0)),
    )
    contrib = pl.pallas_call(
        _group_body,
        grid_spec=grid_spec,
        out_shape=jax.ShapeDtypeStruct((NPAD, H), jnp.float32),
    )(tile_expert, xs, w1_eff, w2_eff, psel, b1, b2)

    CH = min(32, rows_w)

    @functools.partial(
        pl.kernel,
        out_type=jax.ShapeDtypeStruct((T, H), jnp.float32),
        mesh=mesh,
        scratch_types=[
            pltpu.VMEM((CH, H), jnp.float32),
            pltpu.VMEM((CH, H), jnp.float32),
            pltpu.VMEM((CH,), jnp.int32),
            pltpu.VMEM((CH,), jnp.int32),
            pltpu.VMEM((CH,), jnp.float32),
            pltpu.VMEM((CH,), jnp.float32),
            pltpu.SemaphoreType.DMA,
        ],
    )
    def combine(contrib_hbm, d0_hbm, d1_hbm, w0_hbm, w1_hbm, out_hbm,
                r0_v, r1_v, i0_v, i1_v, w0_v, w1_v, sem):
        wid = lax.axis_index("s") * NC + lax.axis_index("c")
        for c in range(rows_w // CH):
            tb = wid * rows_w + c * CH
            pltpu.sync_copy(d0_hbm.at[pl.ds(tb, CH)], i0_v)
            pltpu.sync_copy(d1_hbm.at[pl.ds(tb, CH)], i1_v)
            pltpu.sync_copy(w0_hbm.at[pl.ds(tb, CH)], w0_v)
            pltpu.sync_copy(w1_hbm.at[pl.ds(tb, CH)], w1_v)
            g0 = pltpu.async_copy(contrib_hbm.at[i0_v], r0_v, sem)
            g1 = pltpu.async_copy(contrib_hbm.at[i1_v], r1_v, sem)
            g0.wait()
            g1.wait()

            for g in range(CH // 16):
                wv0 = w0_v[pl.ds(g * 16, 16)]
                wv1 = w1_v[pl.ds(g * 16, 16)]
                for tl in range(16):
                    t = g * 16 + tl
                    wt0 = wv0[tl]
                    wt1 = wv1[tl]

                    def vec_body(j, _, t=t, wt0=wt0, wt1=wt1):
                        sl = pl.ds(j * 16, 16)
                        r0_v[t, sl] = wt0 * r0_v[t, sl] + wt1 * r1_v[t, sl]
                        return 0

                    lax.fori_loop(0, H // 16, vec_body, 0)
            pltpu.sync_copy(r0_v, out_hbm.at[pl.ds(tb, CH)])

    out = combine(contrib, d0, d1, rw0, rw1)
    return out.reshape(B_SZ, S, H)
